# baseline (speedup 1.0000x reference)
import dataclasses
import functools

import jax
import jax.numpy as jnp
from jax import lax
from jax.experimental import pallas as pl
from jax.experimental.pallas import tpu as pltpu
from jax.experimental.pallas import tpu_sc as plsc

N = 10000
E = 160000
D = 256

NC = 2
NS = 16
NT = NC * NS
EP = E // NT
RNG = 312
ACCR = 328
HISTR = 336
CHUNK = 48
BCAP = 768
L = 16

ROWS_BLK = 1000

_mesh = plsc.VectorSubcoreMesh(core_axis_name="c", subcore_axis_name="s")

_sc_params = pltpu.CompilerParams()
if "needs_layout_passes" in pltpu.CompilerParams.__dataclass_fields__:
    _sc_params = dataclasses.replace(_sc_params, needs_layout_passes=False)


@functools.partial(
    pl.kernel,
    out_type=[
        jax.ShapeDtypeStruct((NT, N), jnp.float32),
        jax.ShapeDtypeStruct((NT * NT * 3 * BCAP,), jnp.int32),
        jax.ShapeDtypeStruct((NT * NT,), jnp.int32),
    ],
    mesh=_mesh,
    compiler_params=_sc_params,
    scratch_types=[
        pltpu.VMEM((EP + 8,), jnp.int32),
        pltpu.VMEM((EP + 8,), jnp.int32),
        pltpu.VMEM((EP + 8,), jnp.float32),
        pltpu.VMEM((N,), jnp.float32),
        pltpu.VMEM((L * HISTR,), jnp.float32),
        pltpu.VMEM((NT * 3 * BCAP,), jnp.int32),
        pltpu.VMEM((NT,), jnp.int32),
        pltpu.SMEM((NT,), jnp.int32),
    ],
)
def _sc_prep(src_hbm, dst_hbm, ew_hbm,
             deg_hbm, ed_hbm, cnt_hbm,
             src_v, dst_v, ew_v, deg_v, hist_v, eb_v, cnt_v,
             ptr_sm):
    c = lax.axis_index("c")
    s = lax.axis_index("s")
    wid = c * NS + s

    base = wid * EP
    pltpu.sync_copy(src_hbm.at[pl.ds(base, EP)], src_v.at[pl.ds(0, EP)])
    pltpu.sync_copy(dst_hbm.at[pl.ds(base, EP)], dst_v.at[pl.ds(0, EP)])
    pltpu.sync_copy(ew_hbm.at[pl.ds(base, EP)], ew_v.at[pl.ds(0, EP)])

    zero16 = jnp.zeros((L,), jnp.float32)
    zi = jnp.zeros((L,), jnp.int32)
    lanes = lax.iota(jnp.int32, L)

    @pl.loop(0, N, step=L)
    def _(i):
        deg_v[pl.ds(i, L)] = zero16

    @pl.loop(0, NT)
    def _(b):
        ptr_sm[b] = 0

    def step(i, _):
        full = i < (EP // L)
        mvalid = jnp.where(full, lanes < L, lanes < (EP - (EP // L) * L))
        sv = src_v[pl.ds(i * L, L)]
        dv = dst_v[pl.ds(i * L, L)]
        wv = ew_v[pl.ds(i * L, L)]
        bkt = jnp.minimum(dv // RNG, NT - 1)
        dl = dv - bkt * RNG
        wvi = plsc.bitcast(wv, jnp.int32)
        for b in range(NT):
            mb = jnp.logical_and(bkt == b, mvalid)
            p = ptr_sm[b]
            rb = b * 3 * BCAP
            plsc.store_compressed(eb_v.at[pl.ds(rb + p, L)], sv, mask=mb)
            plsc.store_compressed(eb_v.at[pl.ds(rb + BCAP + p, L)], dl, mask=mb)
            plsc.store_compressed(eb_v.at[pl.ds(rb + 2 * BCAP + p, L)], wvi,
                                  mask=mb)
            ptr_sm[b] = p + plsc.all_reduce_population_count(mb)[0]
        return 0

    nsteps = (EP + L - 1) // L
    lax.fori_loop(0, nsteps, step, 0)

    @pl.loop(0, NT)
    def _(b):
        p = ptr_sm[b]
        rb = b * 3 * BCAP
        for j in range(CHUNK // L):
            eb_v[pl.ds(rb + p + j * L, L)] = wid * RNG + j * L + lanes
            eb_v[pl.ds(rb + BCAP + p + j * L, L)] = zi
            eb_v[pl.ds(rb + 2 * BCAP + p + j * L, L)] = zi
        ptr_sm[b] = ((p + CHUNK - 1) // CHUNK) * CHUNK

    for half in range(2):
        acc = jnp.zeros((L,), jnp.int32)
        for j in range(L):
            acc = acc + jnp.where(lanes == j, ptr_sm[half * L + j], 0)
        cnt_v[pl.ds(half * L, L)] = acc

    @pl.loop(0, NT)
    def _(b):
        @pl.loop(0, L * HISTR, step=L)
        def _(i):
            hist_v[pl.ds(i, L)] = zero16

        def hstep(i, _):
            rb = b * 3 * BCAP
            dl = eb_v[pl.ds(rb + BCAP + i * L, L)]
            wv = plsc.bitcast(eb_v[pl.ds(rb + 2 * BCAP + i * L, L)],
                              jnp.float32)
            plsc.addupdate_scatter(hist_v, [lanes * HISTR + dl], wv)
            return 0

        lax.fori_loop(0, ptr_sm[b] // L, hstep, 0)

        @pl.loop(0, 20)
        def _(g):
            tot = hist_v[pl.ds(g * L, L)]
            for l in range(1, L):
                tot = tot + hist_v[pl.ds(l * HISTR + g * L, L)]
            deg_v[pl.ds(b * RNG + g * L, L)] = tot

        @pl.when(b == NT - 1)
        def _():
            tot = hist_v[pl.ds(RNG, L)]
            for l in range(1, L):
                tot = tot + hist_v[pl.ds(l * HISTR + RNG, L)]
            deg_v[pl.ds(N - L, L)] = tot

    pltpu.sync_copy(deg_v, deg_hbm.at[wid])
    pltpu.sync_copy(eb_v, ed_hbm.at[pl.ds(wid * NT * 3 * BCAP, NT * 3 * BCAP)])
    pltpu.sync_copy(cnt_v, cnt_hbm.at[pl.ds(wid * NT, NT)])


@functools.partial(
    pl.kernel,
    out_type=jax.ShapeDtypeStruct((N, D), jnp.float32),
    mesh=_mesh,
    compiler_params=_sc_params,
    scratch_types=[
        pltpu.VMEM((ACCR, D), jnp.float32),
        pltpu.VMEM((3 * BCAP,), jnp.int32),
        pltpu.VMEM((N,), jnp.float32),
        pltpu.VMEM((CHUNK, D), jnp.float32),
        pltpu.VMEM((CHUNK, D), jnp.float32),
        pltpu.VMEM((CHUNK,), jnp.float32),
        pltpu.VMEM((NT * NT,), jnp.int32),
        pltpu.SemaphoreType.DMA,
        pltpu.SemaphoreType.DMA,
    ],
)
def _sc_agg(h_hbm, ed_hbm, cnt_hbm, dinv_hbm,
            out_hbm,
            acc_v, ed_l, dinv_v, rows0, rows1, norm_v, cnt_v, sem0, sem1):
    c = lax.axis_index("c")
    s = lax.axis_index("s")
    wid = c * NS + s

    pltpu.sync_copy(dinv_hbm, dinv_v)
    pltpu.sync_copy(cnt_hbm, cnt_v)

    zero16 = jnp.zeros((L,), jnp.float32)
    lanes = lax.iota(jnp.int32, L)
    colv = [lanes + j * L for j in range(D // L)]

    @pl.loop(0, ACCR)
    def _(r):
        for j in range(D // L):
            acc_v[r, pl.ds(j * L, L)] = zero16

    dbias = wid * RNG

    def issue(k, rows, sem):
        pltpu.async_copy(h_hbm.at[ed_l.at[pl.ds(k * CHUNK, CHUNK)]], rows, sem)

    def wait(rows, sem):
        pltpu.make_async_copy(h_hbm.at[pl.ds(0, CHUNK)], rows, sem).wait()

    def compute(k, rows):
        ebase = k * CHUNK
        for j in range(CHUNK // L):
            sv = ed_l[pl.ds(ebase + j * L, L)]
            dv = ed_l[pl.ds(BCAP + ebase + j * L, L)]
            wv = plsc.bitcast(ed_l[pl.ds(2 * BCAP + ebase + j * L, L)],
                              jnp.float32)
            nv = (plsc.load_gather(dinv_v, [sv]) * wv *
                  plsc.load_gather(dinv_v, [dv + dbias]))
            norm_v[pl.ds(j * L, L)] = nv

        @plsc.parallel_loop(0, CHUNK, 1, unroll=8)
        def _(r):
            nsplat = plsc.load_gather(norm_v, [jnp.full((L,), r, jnp.int32)])
            rsplat = plsc.load_gather(ed_l, [jnp.full((L,), BCAP + ebase + r,
                                                      jnp.int32)])
            for j in range(D // L):
                val = rows[r, pl.ds(j * L, L)] * nsplat
                plsc.addupdate_scatter(acc_v, [rsplat, colv[j]], val)

    @pl.loop(0, NT)
    def _(t):
        pltpu.sync_copy(
            ed_hbm.at[pl.ds((t * NT + wid) * 3 * BCAP, 3 * BCAP)], ed_l)
        win = cnt_v[pl.ds(t * NT + c * NS, L)]
        n_edges = jnp.max(jnp.where(lanes == s, win, 0))
        nch = n_edges // CHUNK
        npair = nch // 2

        @pl.when(nch > 0)
        def _():
            issue(0, rows0, sem0)

            def pair_body(p, _):
                k0 = 2 * p

                @pl.when(k0 + 1 < nch)
                def _():
                    issue(k0 + 1, rows1, sem1)

                wait(rows0, sem0)
                compute(k0, rows0)

                @pl.when(k0 + 2 < nch)
                def _():
                    issue(k0 + 2, rows0, sem0)

                @pl.when(k0 + 1 < nch)
                def _():
                    wait(rows1, sem1)
                    compute(k0 + 1, rows1)

                return 0

            lax.fori_loop(0, (nch + 1) // 2, pair_body, 0)

    pltpu.sync_copy(acc_v.at[pl.ds(0, RNG)],
                    out_hbm.at[pl.ds(wid * RNG, RNG)])

    @pl.when(wid == NT - 1)
    def _():
        pltpu.sync_copy(acc_v.at[pl.ds(RNG, ACCR - RNG)],
                        out_hbm.at[pl.ds(NT * RNG, N - NT * RNG)])


def _deg_body(degp_ref, o_ref):
    o_ref[...] = lax.rsqrt(jnp.sum(degp_ref[...], axis=0, keepdims=True) + 1.0)


def _dinv_from_partials(deg_p):
    return pl.pallas_call(
        _deg_body,
        grid=(1,),
        in_specs=[pl.BlockSpec((NT, N), lambda i: (0, 0))],
        out_specs=pl.BlockSpec((1, N), lambda i: (0, 0)),
        out_shape=jax.ShapeDtypeStruct((1, N), jnp.float32),
    )(deg_p)


def _mm_body(x_ref, w_ref, o_ref):
    o_ref[...] = jnp.dot(x_ref[...], w_ref[...],
                         preferred_element_type=jnp.float32)


def _matmul(x, w):
    return pl.pallas_call(
        _mm_body,
        grid=(N // ROWS_BLK,),
        in_specs=[
            pl.BlockSpec((ROWS_BLK, D), lambda i: (i, 0)),
            pl.BlockSpec((D, D), lambda i: (0, 0)),
        ],
        out_specs=pl.BlockSpec((ROWS_BLK, D), lambda i: (i, 0)),
        out_shape=jax.ShapeDtypeStruct((N, D), jnp.float32),
    )(x, w)


def _epi_mm_body(acc_ref, h_ref, dinv_ref, b_ref, w_ref, o_ref):
    d2 = dinv_ref[...] * dinv_ref[...]
    z = jnp.maximum(acc_ref[...] + d2 * h_ref[...] + b_ref[...], 0.0)
    o_ref[...] = jnp.dot(z, w_ref[...], preferred_element_type=jnp.float32)


def _epilogue_matmul(acc, h, dinv_col, b_row, w):
    return pl.pallas_call(
        _epi_mm_body,
        grid=(N // ROWS_BLK,),
        in_specs=[
            pl.BlockSpec((ROWS_BLK, D), lambda i: (i, 0)),
            pl.BlockSpec((ROWS_BLK, D), lambda i: (i, 0)),
            pl.BlockSpec((ROWS_BLK, 1), lambda i: (i, 0)),
            pl.BlockSpec((1, D), lambda i: (0, 0)),
            pl.BlockSpec((D, D), lambda i: (0, 0)),
        ],
        out_specs=pl.BlockSpec((ROWS_BLK, D), lambda i: (i, 0)),
        out_shape=jax.ShapeDtypeStruct((N, D), jnp.float32),
    )(acc, h, dinv_col, b_row, w)


def _epi_body(acc_ref, h_ref, dinv_ref, b_ref, o_ref):
    d2 = dinv_ref[...] * dinv_ref[...]
    o_ref[...] = jnp.maximum(acc_ref[...] + d2 * h_ref[...] + b_ref[...], 0.0)


def _epilogue(acc, h, dinv_col, b_row):
    return pl.pallas_call(
        _epi_body,
        grid=(N // ROWS_BLK,),
        in_specs=[
            pl.BlockSpec((ROWS_BLK, D), lambda i: (i, 0)),
            pl.BlockSpec((ROWS_BLK, D), lambda i: (i, 0)),
            pl.BlockSpec((ROWS_BLK, 1), lambda i: (i, 0)),
            pl.BlockSpec((1, D), lambda i: (0, 0)),
        ],
        out_specs=pl.BlockSpec((ROWS_BLK, D), lambda i: (i, 0)),
        out_shape=jax.ShapeDtypeStruct((N, D), jnp.float32),
    )(acc, h, dinv_col, b_row)


def kernel(x, edge_index, edge_weight, W1, b1, W2, b2):
    src = edge_index[0]
    dst = edge_index[1]

    deg_p, edata, cnt = _sc_prep(src, dst, edge_weight)
    dinv_row = _dinv_from_partials(deg_p)
    dinv = dinv_row.reshape(N)
    dinv_col = dinv_row.reshape(N, 1)
    b1_row = b1[None, :]
    b2_row = b2[None, :]

    h1 = _matmul(x, W1)
    acc1 = _sc_agg(h1, edata, cnt, dinv)
    h2 = _epilogue_matmul(acc1, h1, dinv_col, b1_row, W2)
    acc2 = _sc_agg(h2, edata, cnt, dinv)
    return _epilogue(acc2, h2, dinv_col, b2_row)


# speedup vs baseline: 1.9033x; 1.9033x over previous
import dataclasses
import functools

import jax
import jax.numpy as jnp
from jax import lax
from jax.experimental import pallas as pl
from jax.experimental.pallas import tpu as pltpu
from jax.experimental.pallas import tpu_sc as plsc

N = 10000
E = 160000
D = 256

NC = 2
NS = 16
NT = NC * NS
EP = E // NT
RNG = 312
ACCR = 328
HISTR = 336
CHUNK = 48
BCAP = 768
L = 16

ROWS_BLK = 1000

_mesh = plsc.VectorSubcoreMesh(core_axis_name="c", subcore_axis_name="s")

_sc_params = pltpu.CompilerParams()
if "needs_layout_passes" in pltpu.CompilerParams.__dataclass_fields__:
    _sc_params = dataclasses.replace(_sc_params, needs_layout_passes=False)


@functools.partial(
    pl.kernel,
    out_type=[
        jax.ShapeDtypeStruct((NT, N), jnp.float32),
        jax.ShapeDtypeStruct((NT * NT * 3 * BCAP,), jnp.int32),
        jax.ShapeDtypeStruct((NT * NT,), jnp.int32),
    ],
    mesh=_mesh,
    compiler_params=_sc_params,
    scratch_types=[
        pltpu.VMEM((EP + 8,), jnp.int32),
        pltpu.VMEM((EP + 8,), jnp.int32),
        pltpu.VMEM((EP + 8,), jnp.float32),
        pltpu.VMEM((N,), jnp.float32),
        pltpu.VMEM((L * HISTR,), jnp.float32),
        pltpu.VMEM((NT * 3 * BCAP,), jnp.int32),
        pltpu.VMEM((NT,), jnp.int32),
        pltpu.VMEM((4 * 3 * 1568,), jnp.int32),
        pltpu.SMEM((NT + 4,), jnp.int32),
    ],
)
def _sc_prep(src_hbm, dst_hbm, ew_hbm,
             deg_hbm, ed_hbm, cnt_hbm,
             src_v, dst_v, ew_v, deg_v, hist_v, eb_v, cnt_v, qb_v,
             ptr_sm):
    c = lax.axis_index("c")
    s = lax.axis_index("s")
    wid = c * NS + s

    base = wid * EP
    pltpu.sync_copy(src_hbm.at[pl.ds(base, EP)], src_v.at[pl.ds(0, EP)])
    pltpu.sync_copy(dst_hbm.at[pl.ds(base, EP)], dst_v.at[pl.ds(0, EP)])
    pltpu.sync_copy(ew_hbm.at[pl.ds(base, EP)], ew_v.at[pl.ds(0, EP)])

    zero16 = jnp.zeros((L,), jnp.float32)
    zi = jnp.zeros((L,), jnp.int32)
    lanes = lax.iota(jnp.int32, L)

    @pl.loop(0, N, step=L)
    def _(i):
        deg_v[pl.ds(i, L)] = zero16

    QCAP = 1568
    QR = 8 * RNG

    @pl.loop(0, NT + 4)
    def _(b):
        ptr_sm[b] = 0

    def step(i, _):
        full = i < (EP // L)
        mvalid = jnp.where(full, lanes < L, lanes < (EP - (EP // L) * L))
        sv = src_v[pl.ds(i * L, L)]
        dv = dst_v[pl.ds(i * L, L)]
        wv = ew_v[pl.ds(i * L, L)]
        qk = jnp.minimum(dv // QR, 3)
        wvi = plsc.bitcast(wv, jnp.int32)
        for q in range(4):
            mq = jnp.logical_and(qk == q, mvalid)
            p = ptr_sm[NT + q]
            rq = q * 3 * QCAP
            plsc.store_compressed(qb_v.at[pl.ds(rq + p, L)], sv, mask=mq)
            plsc.store_compressed(qb_v.at[pl.ds(rq + QCAP + p, L)], dv,
                                  mask=mq)
            plsc.store_compressed(qb_v.at[pl.ds(rq + 2 * QCAP + p, L)], wvi,
                                  mask=mq)
            ptr_sm[NT + q] = p + plsc.all_reduce_population_count(mq)[0]
        return 0

    nsteps = (EP + L - 1) // L
    lax.fori_loop(0, nsteps, step, 0)

    for q in range(4):
        nq = ptr_sm[NT + q]
        rq = q * 3 * QCAP

        def step2(i, _):
            mvalid = lanes < (nq - i * L)
            sv = qb_v[pl.ds(rq + i * L, L)]
            dv = qb_v[pl.ds(rq + QCAP + i * L, L)]
            wvi = qb_v[pl.ds(rq + 2 * QCAP + i * L, L)]
            bkt = jnp.minimum(dv // RNG, NT - 1)
            dl = dv - bkt * RNG
            for b in range(q * 8, q * 8 + 8):
                mb = jnp.logical_and(bkt == b, mvalid)
                p = ptr_sm[b]
                rb = b * 3 * BCAP
                plsc.store_compressed(eb_v.at[pl.ds(rb + p, L)], sv, mask=mb)
                plsc.store_compressed(eb_v.at[pl.ds(rb + BCAP + p, L)], dl,
                                      mask=mb)
                plsc.store_compressed(eb_v.at[pl.ds(rb + 2 * BCAP + p, L)],
                                      wvi, mask=mb)
                ptr_sm[b] = p + plsc.all_reduce_population_count(mb)[0]
            return 0

        lax.fori_loop(0, (nq + L - 1) // L, step2, 0)

    @pl.loop(0, NT)
    def _(b):
        p = ptr_sm[b]
        rb = b * 3 * BCAP
        for j in range(CHUNK // L):
            eb_v[pl.ds(rb + p + j * L, L)] = wid * RNG + j * L + lanes
            eb_v[pl.ds(rb + BCAP + p + j * L, L)] = zi
            eb_v[pl.ds(rb + 2 * BCAP + p + j * L, L)] = zi
        ptr_sm[b] = ((p + CHUNK - 1) // CHUNK) * CHUNK

    for half in range(2):
        acc = jnp.zeros((L,), jnp.int32)
        for j in range(L):
            acc = acc + jnp.where(lanes == j, ptr_sm[half * L + j], 0)
        cnt_v[pl.ds(half * L, L)] = acc

    @pl.loop(0, NT)
    def _(b):
        @pl.loop(0, L * HISTR, step=L)
        def _(i):
            hist_v[pl.ds(i, L)] = zero16

        def hstep(i, _):
            rb = b * 3 * BCAP
            dl = eb_v[pl.ds(rb + BCAP + i * L, L)]
            wv = plsc.bitcast(eb_v[pl.ds(rb + 2 * BCAP + i * L, L)],
                              jnp.float32)
            plsc.addupdate_scatter(hist_v, [lanes * HISTR + dl], wv)
            return 0

        lax.fori_loop(0, ptr_sm[b] // L, hstep, 0)

        @pl.loop(0, 20)
        def _(g):
            tot = hist_v[pl.ds(g * L, L)]
            for l in range(1, L):
                tot = tot + hist_v[pl.ds(l * HISTR + g * L, L)]
            deg_v[pl.ds(b * RNG + g * L, L)] = tot

        @pl.when(b == NT - 1)
        def _():
            tot = hist_v[pl.ds(RNG, L)]
            for l in range(1, L):
                tot = tot + hist_v[pl.ds(l * HISTR + RNG, L)]
            deg_v[pl.ds(N - L, L)] = tot

    pltpu.sync_copy(deg_v, deg_hbm.at[wid])
    pltpu.sync_copy(eb_v, ed_hbm.at[pl.ds(wid * NT * 3 * BCAP, NT * 3 * BCAP)])
    pltpu.sync_copy(cnt_v, cnt_hbm.at[pl.ds(wid * NT, NT)])


@functools.partial(
    pl.kernel,
    out_type=jax.ShapeDtypeStruct((N, D), jnp.float32),
    mesh=_mesh,
    compiler_params=_sc_params,
    scratch_types=[
        pltpu.VMEM((ACCR, D), jnp.float32),
        pltpu.VMEM((3 * BCAP,), jnp.int32),
        pltpu.VMEM((N,), jnp.float32),
        pltpu.VMEM((CHUNK, D), jnp.float32),
        pltpu.VMEM((CHUNK, D), jnp.float32),
        pltpu.VMEM((CHUNK,), jnp.float32),
        pltpu.VMEM((NT * NT,), jnp.int32),
        pltpu.SemaphoreType.DMA,
        pltpu.SemaphoreType.DMA,
    ],
)
def _sc_agg(h_hbm, ed_hbm, cnt_hbm, dinv_hbm,
            out_hbm,
            acc_v, ed_l, dinv_v, rows0, rows1, norm_v, cnt_v, sem0, sem1):
    c = lax.axis_index("c")
    s = lax.axis_index("s")
    wid = c * NS + s

    pltpu.sync_copy(dinv_hbm, dinv_v)
    pltpu.sync_copy(cnt_hbm, cnt_v)

    zero16 = jnp.zeros((L,), jnp.float32)
    lanes = lax.iota(jnp.int32, L)
    colv = [lanes + j * L for j in range(D // L)]

    @pl.loop(0, ACCR)
    def _(r):
        for j in range(D // L):
            acc_v[r, pl.ds(j * L, L)] = zero16

    dbias = wid * RNG

    def issue(k, rows, sem):
        pltpu.async_copy(h_hbm.at[ed_l.at[pl.ds(k * CHUNK, CHUNK)]], rows, sem)

    def wait(rows, sem):
        pltpu.make_async_copy(h_hbm.at[pl.ds(0, CHUNK)], rows, sem).wait()

    def compute(k, rows):
        ebase = k * CHUNK
        for j in range(CHUNK // L):
            sv = ed_l[pl.ds(ebase + j * L, L)]
            dv = ed_l[pl.ds(BCAP + ebase + j * L, L)]
            wv = plsc.bitcast(ed_l[pl.ds(2 * BCAP + ebase + j * L, L)],
                              jnp.float32)
            nv = (plsc.load_gather(dinv_v, [sv]) * wv *
                  plsc.load_gather(dinv_v, [dv + dbias]))
            norm_v[pl.ds(j * L, L)] = nv

        @plsc.parallel_loop(0, CHUNK, 1, unroll=4)
        def _(r):
            nsplat = plsc.load_gather(norm_v, [jnp.full((L,), r, jnp.int32)])
            rsplat = plsc.load_gather(ed_l, [jnp.full((L,), BCAP + ebase + r,
                                                      jnp.int32)])
            for j in range(D // L):
                val = rows[r, pl.ds(j * L, L)] * nsplat
                plsc.addupdate_scatter(acc_v, [rsplat, colv[j]], val)

    @pl.loop(0, NT)
    def _(t):
        pltpu.sync_copy(
            ed_hbm.at[pl.ds((t * NT + wid) * 3 * BCAP, 3 * BCAP)], ed_l)
        win = cnt_v[pl.ds(t * NT + c * NS, L)]
        n_edges = jnp.max(jnp.where(lanes == s, win, 0))
        nch = n_edges // CHUNK
        npair = nch // 2

        @pl.when(nch > 0)
        def _():
            issue(0, rows0, sem0)

            def pair_body(p, _):
                k0 = 2 * p

                @pl.when(k0 + 1 < nch)
                def _():
                    issue(k0 + 1, rows1, sem1)

                wait(rows0, sem0)
                compute(k0, rows0)

                @pl.when(k0 + 2 < nch)
                def _():
                    issue(k0 + 2, rows0, sem0)

                @pl.when(k0 + 1 < nch)
                def _():
                    wait(rows1, sem1)
                    compute(k0 + 1, rows1)

                return 0

            lax.fori_loop(0, (nch + 1) // 2, pair_body, 0)

    pltpu.sync_copy(acc_v.at[pl.ds(0, RNG)],
                    out_hbm.at[pl.ds(wid * RNG, RNG)])

    @pl.when(wid == NT - 1)
    def _():
        pltpu.sync_copy(acc_v.at[pl.ds(RNG, ACCR - RNG)],
                        out_hbm.at[pl.ds(NT * RNG, N - NT * RNG)])


def _deg_body(degp_ref, o_ref):
    o_ref[...] = lax.rsqrt(jnp.sum(degp_ref[...], axis=0, keepdims=True) + 1.0)


def _dinv_from_partials(deg_p):
    return pl.pallas_call(
        _deg_body,
        grid=(1,),
        in_specs=[pl.BlockSpec((NT, N), lambda i: (0, 0))],
        out_specs=pl.BlockSpec((1, N), lambda i: (0, 0)),
        out_shape=jax.ShapeDtypeStruct((1, N), jnp.float32),
    )(deg_p)


def _mm_body(x_ref, w_ref, o_ref):
    o_ref[...] = jnp.dot(x_ref[...], w_ref[...],
                         preferred_element_type=jnp.float32)


def _matmul(x, w):
    return pl.pallas_call(
        _mm_body,
        grid=(N // ROWS_BLK,),
        in_specs=[
            pl.BlockSpec((ROWS_BLK, D), lambda i: (i, 0)),
            pl.BlockSpec((D, D), lambda i: (0, 0)),
        ],
        out_specs=pl.BlockSpec((ROWS_BLK, D), lambda i: (i, 0)),
        out_shape=jax.ShapeDtypeStruct((N, D), jnp.float32),
    )(x, w)


def _epi_mm_body(acc_ref, h_ref, dinv_ref, b_ref, w_ref, o_ref):
    d2 = dinv_ref[...] * dinv_ref[...]
    z = jnp.maximum(acc_ref[...] + d2 * h_ref[...] + b_ref[...], 0.0)
    o_ref[...] = jnp.dot(z, w_ref[...], preferred_element_type=jnp.float32)


def _epilogue_matmul(acc, h, dinv_col, b_row, w):
    return pl.pallas_call(
        _epi_mm_body,
        grid=(N // ROWS_BLK,),
        in_specs=[
            pl.BlockSpec((ROWS_BLK, D), lambda i: (i, 0)),
            pl.BlockSpec((ROWS_BLK, D), lambda i: (i, 0)),
            pl.BlockSpec((ROWS_BLK, 1), lambda i: (i, 0)),
            pl.BlockSpec((1, D), lambda i: (0, 0)),
            pl.BlockSpec((D, D), lambda i: (0, 0)),
        ],
        out_specs=pl.BlockSpec((ROWS_BLK, D), lambda i: (i, 0)),
        out_shape=jax.ShapeDtypeStruct((N, D), jnp.float32),
    )(acc, h, dinv_col, b_row, w)


def _epi_body(acc_ref, h_ref, dinv_ref, b_ref, o_ref):
    d2 = dinv_ref[...] * dinv_ref[...]
    o_ref[...] = jnp.maximum(acc_ref[...] + d2 * h_ref[...] + b_ref[...], 0.0)


def _epilogue(acc, h, dinv_col, b_row):
    return pl.pallas_call(
        _epi_body,
        grid=(N // ROWS_BLK,),
        in_specs=[
            pl.BlockSpec((ROWS_BLK, D), lambda i: (i, 0)),
            pl.BlockSpec((ROWS_BLK, D), lambda i: (i, 0)),
            pl.BlockSpec((ROWS_BLK, 1), lambda i: (i, 0)),
            pl.BlockSpec((1, D), lambda i: (0, 0)),
        ],
        out_specs=pl.BlockSpec((ROWS_BLK, D), lambda i: (i, 0)),
        out_shape=jax.ShapeDtypeStruct((N, D), jnp.float32),
    )(acc, h, dinv_col, b_row)


def kernel(x, edge_index, edge_weight, W1, b1, W2, b2):
    src = edge_index[0]
    dst = edge_index[1]

    deg_p, edata, cnt = _sc_prep(src, dst, edge_weight)
    dinv_row = _dinv_from_partials(deg_p)
    dinv = dinv_row.reshape(N)
    dinv_col = dinv_row.reshape(N, 1)
    b1_row = b1[None, :]
    b2_row = b2[None, :]

    h1 = _matmul(x, W1)
    acc1 = _sc_agg(h1, edata, cnt, dinv)
    h2 = _epilogue_matmul(acc1, h1, dinv_col, b1_row, W2)
    acc2 = _sc_agg(h2, edata, cnt, dinv)
    return _epilogue(acc2, h2, dinv_col, b2_row)
